# baseline (speedup 1.0000x reference)
import functools
import jax
import jax.numpy as jnp
from jax import lax
from jax.experimental import pallas as pl
from jax.experimental.pallas import tpu as pltpu, tpu_sc as plsc

EMB_DIM = 64
SCALE = 8.0

NC = 2
NS = 16
NW = NC * NS
CH = 128
NBUF = 2


@functools.partial(jax.jit, static_argnames=("nj",))
def _emb_lookup(table, x5, nj):
    @functools.partial(
        pl.kernel,
        out_type=jax.ShapeDtypeStruct((nj, 8, NW, 8, CH), jnp.float32),
        mesh=plsc.VectorSubcoreMesh(
            core_axis_name="c", subcore_axis_name="s",
            num_cores=NC, num_subcores=NS,
        ),
        scratch_types=(
            [pltpu.VMEM((nj // 8, 8, CH), jnp.int32)]
            + [pltpu.VMEM((NBUF, CH, EMB_DIM), jnp.float32)]
            + [pltpu.VMEM((NBUF, 8, 8, CH), jnp.float32)]
            + [pltpu.SemaphoreType.DMA] * (2 * NBUF)
        ),
        compiler_params=pltpu.CompilerParams(
            use_tc_tiling_on_sc=False, needs_layout_passes=False),
    )
    def body(table_hbm, x5_hbm, out_hbm, idx_v, in_v, out_v, *sems):
        gsem = sems[:NBUF]
        ssem = sems[NBUF:]
        wid = lax.axis_index("s") * NC + lax.axis_index("c")
        pltpu.sync_copy(x5_hbm.at[:, wid], idx_v)

        def start_gather(j, b):
            pltpu.async_copy(
                table_hbm.at[idx_v.at[j // 8, j % 8]], in_v.at[b], gsem[b])

        def process(j, b, fetch, first):
            pltpu.make_async_copy(
                table_hbm.at[idx_v.at[j // 8, j % 8]], in_v.at[b],
                gsem[b]).wait()
            if not first:
                pltpu.make_async_copy(
                    out_v.at[b], out_hbm.at[j, :, wid], ssem[b]).wait()

            bvec = jnp.full((16,), b, jnp.int32)
            iot = lax.iota(jnp.int32, 16)

            @plsc.parallel_loop(0, CH, step=16)
            def _t(t):
                rvec = t + iot
                for ch in range(8):
                    for cl in range(8):
                        cvec = jnp.full((16,), ch * 8 + cl, jnp.int32)
                        v = plsc.load_gather(in_v, [bvec, rvec, cvec])
                        out_v[b, ch, cl, pl.ds(t, 16)] = v * SCALE

            pltpu.async_copy(out_v.at[b], out_hbm.at[j, :, wid], ssem[b])
            if fetch:
                start_gather(j + NBUF, b)

        for b in range(NBUF):
            start_gather(b, b)
        for b in range(NBUF):
            process(b, b, fetch=True, first=True)

        @pl.loop(0, (nj - 2 * NBUF) // NBUF)
        def _main(s):
            j0 = NBUF + NBUF * s
            for db in range(NBUF):
                process(j0 + db, db, fetch=True, first=False)

        for db in range(NBUF):
            process(nj - NBUF + db, db, fetch=False, first=False)
        for b in range(NBUF):
            pltpu.make_async_copy(
                out_v.at[b], out_hbm.at[nj - NBUF + b, :, wid],
                ssem[b]).wait()

    return body(table, x5)


def kernel(x, table):
    ntok, nj = x.shape
    assert ntok == NW * CH and nj % 8 == 0
    xi = x.astype(jnp.int32)
    x5 = xi.reshape(NW, CH, nj // 8, 8).transpose(2, 0, 3, 1)
    out5 = _emb_lookup(table, x5, nj)
    return out5.transpose(2, 4, 0, 1, 3).reshape(ntok, nj, EMB_DIM)


# speedup vs baseline: 1.2321x; 1.2321x over previous
import functools
import jax
import jax.numpy as jnp
from jax import lax
from jax.experimental import pallas as pl
from jax.experimental.pallas import tpu as pltpu, tpu_sc as plsc

EMB_DIM = 64
SCALE = 8.0

NC = 2
NS = 16
NW = NC * NS
CH = 128
NBUF = 2


@functools.partial(jax.jit, static_argnames=("nj",))
def _emb_lookup(table, x5, nj):
    @functools.partial(
        pl.kernel,
        out_type=jax.ShapeDtypeStruct((nj, 8, NW, 8, CH), jnp.float32),
        mesh=plsc.VectorSubcoreMesh(
            core_axis_name="c", subcore_axis_name="s",
            num_cores=NC, num_subcores=NS,
        ),
        scratch_types=(
            [pltpu.VMEM((nj // 8, 8, CH), jnp.int32)]
            + [pltpu.VMEM((NBUF, CH, EMB_DIM), jnp.float32)]
            + [pltpu.VMEM((NBUF, 8, 8, CH), jnp.float32)]
            + [pltpu.SemaphoreType.DMA] * (2 * NBUF)
        ),
        compiler_params=pltpu.CompilerParams(
            use_tc_tiling_on_sc=False, needs_layout_passes=False),
    )
    def body(table_hbm, x5_hbm, out_hbm, idx_v, in_v, out_v, *sems):
        gsem = sems[:NBUF]
        ssem = sems[NBUF:]
        wid = lax.axis_index("s") * NC + lax.axis_index("c")
        pltpu.sync_copy(x5_hbm.at[:, wid], idx_v)

        def start_gather(j, b):
            pltpu.async_copy(
                table_hbm.at[idx_v.at[j // 8, j % 8]], in_v.at[b], gsem[b])

        iot = lax.iota(jnp.int32, 16)
        pks = [(iot + k) & 15 for k in range(16)]
        chs = [pk >> 3 for pk in pks]
        cls = [pk & 7 for pk in pks]

        for b in range(NBUF):
            start_gather(b, b)

        @pl.loop(0, nj // NBUF)
        def _main(s):
            j0 = NBUF * s
            for b in range(NBUF):
                j = j0 + b
                pltpu.make_async_copy(
                    table_hbm.at[idx_v.at[j // 8, j % 8]], in_v.at[b],
                    gsem[b]).wait()

                @pl.when(j >= NBUF)
                def _wait_prev():
                    pltpu.make_async_copy(
                        out_v.at[b], out_hbm.at[j, :, wid], ssem[b]).wait()

                bvec = jnp.full((16,), b, jnp.int32)

                @plsc.parallel_loop(0, CH, step=16)
                def _t(t):
                    rvec = t + iot

                    @pl.loop(0, EMB_DIM // 16)
                    def _tc(tc):
                        for k in range(16):
                            cv = tc * 16 + pks[k]
                            v = plsc.load_gather(in_v, [bvec, rvec, cv])
                            plsc.store_scatter(
                                out_v,
                                [bvec, (tc * 2) + chs[k], cls[k], rvec],
                                v * SCALE)

                pltpu.async_copy(out_v.at[b], out_hbm.at[j, :, wid], ssem[b])

                @pl.when(j + NBUF < nj)
                def _fetch_next():
                    start_gather(j + NBUF, b)

        for b in range(NBUF):
            pltpu.make_async_copy(
                out_v.at[b], out_hbm.at[nj - NBUF + b, :, wid],
                ssem[b]).wait()

    return body(table, x5)


def kernel(x, table):
    ntok, nj = x.shape
    assert ntok == NW * CH and nj % 8 == 0
    xi = x.astype(jnp.int32)
    x5 = xi.reshape(NW, CH, nj // 8, 8).transpose(2, 0, 3, 1)
    out5 = _emb_lookup(table, x5, nj)
    return out5.transpose(2, 4, 0, 1, 3).reshape(ntok, nj, EMB_DIM)


# speedup vs baseline: 1.2823x; 1.0408x over previous
import functools
import jax
import jax.numpy as jnp
from jax import lax
from jax.experimental import pallas as pl
from jax.experimental.pallas import tpu as pltpu, tpu_sc as plsc

EMB_DIM = 64
SCALE = 8.0

NC = 2
NS = 16
NW = NC * NS
CH = 128
NBUF = 2


@functools.partial(jax.jit, static_argnames=("nj",))
def _emb_lookup(table, x5, nj):
    @functools.partial(
        pl.kernel,
        out_type=jax.ShapeDtypeStruct((nj, 8, NW, 8, CH), jnp.float32),
        mesh=plsc.VectorSubcoreMesh(
            core_axis_name="c", subcore_axis_name="s",
            num_cores=NC, num_subcores=NS,
        ),
        scratch_types=(
            [pltpu.VMEM((nj // 8, 8, CH), jnp.int32)]
            + [pltpu.VMEM((NBUF, CH, EMB_DIM), jnp.float32)]
            + [pltpu.VMEM((NBUF, 8, 8, CH), jnp.float32)]
            + [pltpu.SemaphoreType.DMA] * (2 * NBUF)
        ),
        compiler_params=pltpu.CompilerParams(
            use_tc_tiling_on_sc=False, needs_layout_passes=False,
            disable_bounds_checks=True),
    )
    def body(table_hbm, x5_hbm, out_hbm, idx_v, in_v, out_v, *sems):
        gsem = sems[:NBUF]
        ssem = sems[NBUF:]
        wid = lax.axis_index("s") * NC + lax.axis_index("c")
        pltpu.sync_copy(x5_hbm.at[:, wid], idx_v)

        def start_gather(j, b):
            pltpu.async_copy(
                table_hbm.at[idx_v.at[j // 8, j % 8]], in_v.at[b], gsem[b])

        iot = lax.iota(jnp.int32, 16)
        zvec = jnp.zeros((16,), jnp.int32)
        pks = [(iot + k) & 15 for k in range(16)]
        pk7s = [pk << 7 for pk in pks]

        for b in range(NBUF):
            start_gather(b, b)

        @pl.loop(0, nj // NBUF)
        def _main(s):
            j0 = NBUF * s
            for b in range(NBUF):
                j = j0 + b
                pltpu.make_async_copy(
                    table_hbm.at[idx_v.at[j // 8, j % 8]], in_v.at[b],
                    gsem[b]).wait()

                @pl.when(j >= NBUF)
                def _wait_prev():
                    pltpu.make_async_copy(
                        out_v.at[b], out_hbm.at[j, :, wid], ssem[b]).wait()

                bvec = jnp.full((16,), b, jnp.int32)

                @plsc.parallel_loop(0, CH, step=16)
                def _t(t):
                    rvec = t + iot
                    rv64 = rvec << 6

                    for tc in range(EMB_DIM // 16):
                        ga = rv64 + tc * 16
                        sa = rvec + tc * 2048
                        for k in range(16):
                            v = plsc.load_gather(
                                in_v, [bvec, zvec, ga + pks[k]])
                            plsc.store_scatter(
                                out_v, [bvec, zvec, zvec, sa + pk7s[k]],
                                v * SCALE)

                pltpu.async_copy(out_v.at[b], out_hbm.at[j, :, wid], ssem[b])

                @pl.when(j + NBUF < nj)
                def _fetch_next():
                    start_gather(j + NBUF, b)

        for b in range(NBUF):
            pltpu.make_async_copy(
                out_v.at[b], out_hbm.at[nj - NBUF + b, :, wid],
                ssem[b]).wait()

    return body(table, x5)


def kernel(x, table):
    ntok, nj = x.shape
    assert ntok == NW * CH and nj % 8 == 0
    xi = x.astype(jnp.int32)
    x5 = xi.reshape(NW, CH, nj // 8, 8).transpose(2, 0, 3, 1)
    out5 = _emb_lookup(table, x5, nj)
    return out5.transpose(2, 4, 0, 1, 3).reshape(ntok, nj, EMB_DIM)
